# baseline (device time: 11227 ns/iter reference)
import jax
import jax.numpy as jnp
from jax import lax
from jax.experimental import pallas as pl
from jax.experimental.pallas import tpu as pltpu

G = 4


def kernel(x):
    m, n = x.shape
    mb = m // G
    pb = mb // 128

    def body(x_ref, out_ref, comm_ref, send_sems, recv_sems):
        i = pl.program_id(0)
        my_x = lax.axis_index("x")
        my_y = lax.axis_index("y")
        nbr = (my_x, 1 - my_y)

        barrier_sem = pltpu.get_barrier_semaphore()

        @pl.when(i == 0)
        def _():
            pl.semaphore_signal(
                barrier_sem, inc=1, device_id=nbr,
                device_id_type=pl.DeviceIdType.MESH,
            )

        s = jnp.sum(x_ref[:, :], axis=1)
        comm_ref[0, pl.ds(i * pb, pb), :] = s.reshape(pb, 128)

        @pl.when(i == 0)
        def _():
            pl.semaphore_wait(barrier_sem, 1)

        def send_step(k):
            rdma = pltpu.make_async_remote_copy(
                src_ref=comm_ref.at[0, pl.ds(k * pb, pb), :],
                dst_ref=comm_ref.at[1, pl.ds(k * pb, pb), :],
                send_sem=send_sems.at[k],
                recv_sem=recv_sems.at[k],
                device_id=nbr,
                device_id_type=pl.DeviceIdType.MESH,
            )
            return rdma

        for k in range(G):
            @pl.when(i == k)
            def _():
                send_step(k).start()

        @pl.when(i == G - 1)
        def _():
            for k in range(G):
                send_step(k).wait()
            out_ref[:, :] = comm_ref[0, :, :] + comm_ref[1, :, :]

    packed = pl.pallas_call(
        body,
        grid=(G,),
        out_shape=jax.ShapeDtypeStruct((m // 128, 128), jnp.float32),
        in_specs=[
            pl.BlockSpec((mb, n), lambda i: (i, 0), memory_space=pltpu.VMEM)
        ],
        out_specs=pl.BlockSpec(
            (m // 128, 128), lambda i: (0, 0), memory_space=pltpu.VMEM
        ),
        scratch_shapes=[
            pltpu.VMEM((2, m // 128, 128), jnp.float32),
            pltpu.SemaphoreType.DMA((G,)),
            pltpu.SemaphoreType.DMA((G,)),
        ],
        compiler_params=pltpu.CompilerParams(collective_id=0),
    )(x)
    return packed.reshape(m, 1)
